# baseline (speedup 1.0000x reference)
import functools

import jax
import jax.numpy as jnp
from jax import lax
from jax.experimental import pallas as pl
from jax.experimental.pallas import tpu as pltpu
from jax.experimental.pallas import tpu_sc as plsc


def _make_gather(total: int, D: int):
    info = plsc.get_sparse_core_info()
    NC, NS = info.num_cores, info.num_subcores
    NW = NC * NS
    assert total % NW == 0
    b_per_w = total // NW
    C = 800
    assert b_per_w % C == 0
    nchunk = b_per_w // C

    mesh = plsc.VectorSubcoreMesh(core_axis_name="c", subcore_axis_name="s")

    @functools.partial(
        pl.kernel,
        mesh=mesh,
        out_type=jax.ShapeDtypeStruct((total, D), jnp.float32),
        scratch_types=[
            pltpu.VMEM((b_per_w,), jnp.int32),
            pltpu.VMEM((2, C, D), jnp.float32),
            pltpu.SemaphoreType.DMA,
            pltpu.SemaphoreType.DMA,
        ],
        compiler_params=pltpu.CompilerParams(use_tc_tiling_on_sc=False),
    )
    def gather_kernel(table_hbm, idx_hbm, out_hbm, idx_v, rows_v, gsem0, gsem1):
        wid = lax.axis_index("s") * NC + lax.axis_index("c")
        base = wid * b_per_w
        pltpu.sync_copy(idx_hbm.at[pl.ds(base, b_per_w)], idx_v)
        gsems = (gsem0, gsem1)
        cp0 = pltpu.async_copy(
            table_hbm.at[idx_v.at[pl.ds(0, C)]], rows_v.at[0], gsems[0])
        copies = [cp0, None]
        for c in range(nchunk):
            buf = c % 2
            if c + 1 < nchunk:
                nbuf = (c + 1) % 2
                copies[nbuf] = pltpu.async_copy(
                    table_hbm.at[idx_v.at[pl.ds((c + 1) * C, C)]],
                    rows_v.at[nbuf], gsems[nbuf])
            copies[buf].wait()
            pltpu.sync_copy(rows_v.at[buf], out_hbm.at[pl.ds(base + c * C, C)])

    return gather_kernel


def kernel(item_ids, item_codes):
    B, H = item_ids.shape
    N, D = item_codes.shape
    total = B * H
    flat_ids = item_ids.reshape(total).astype(jnp.int32)
    one = (1 - jnp.minimum(flat_ids[0], 0)).astype(jnp.float32)
    out = _make_gather(total, D)(item_codes * one, flat_ids)
    return (out * one).reshape(B, H, D)


# speedup vs baseline: 1.0669x; 1.0669x over previous
import functools

import jax
import jax.numpy as jnp
from jax import lax
from jax.experimental import pallas as pl
from jax.experimental.pallas import tpu as pltpu
from jax.experimental.pallas import tpu_sc as plsc


def _sc_mesh():
    info = plsc.get_sparse_core_info()
    NC, NS = info.num_cores, info.num_subcores
    mesh = plsc.VectorSubcoreMesh(core_axis_name="c", subcore_axis_name="s")
    return NC, NS, mesh


def _make_repack(N: int, D: int):
    NC, NS, mesh = _sc_mesh()
    NW = NC * NS
    K = 400
    assert N % K == 0 and K % 16 == 0
    nchunks = N // K
    main = (nchunks // NW) & ~1
    extra = nchunks - main * NW

    @functools.partial(
        pl.kernel,
        mesh=mesh,
        out_type=jax.ShapeDtypeStruct((N // 2, 2 * D), jnp.float32),
        scratch_types=[
            pltpu.VMEM((2, K, D), jnp.float32),
            pltpu.VMEM((K // 2, 2 * D), jnp.float32),
            pltpu.SemaphoreType.DMA,
            pltpu.SemaphoreType.DMA,
        ],
    )
    def repack_kernel(table_hbm, out_hbm, buf, pbuf, s0, s1):
        wid = lax.axis_index("s") * NC + lax.axis_index("c")
        sems = (s0, s1)

        def start(cid, b):
            pltpu.async_copy(
                table_hbm.at[pl.ds(cid * K, K), :], buf.at[b], sems[b])

        def finish(cid, b):
            pltpu.make_async_copy(
                table_hbm.at[pl.ds(0, K), :], buf.at[b], sems[b]).wait()

            def pack(j, _):
                for p in range(2 * D // 16):
                    pbuf[j, pl.ds(p * 16, 16)] = (
                        buf[b, 2 * j + p * 16 // D, pl.ds(p * 16 % D, 16)])
                return ()
            lax.fori_loop(0, K // 2, pack, ())
            pltpu.sync_copy(
                pbuf, out_hbm.at[pl.ds(cid * (K // 2), K // 2), :])

        start(wid, 0)

        def pair_body(i2, _):
            for b in range(2):
                i = i2 * 2 + b

                @pl.when(i + 1 < main)
                def _():
                    start(wid + (i + 1) * NW, (b + 1) % 2)

                finish(wid + i * NW, b)
            return ()

        lax.fori_loop(0, main // 2, pair_body, ())

        @pl.when(wid < extra)
        def _():
            start(main * NW + wid, 0)
            finish(main * NW + wid, 0)

    return repack_kernel


def _make_pair_gather(total: int, N: int, D: int):
    NC, NS, mesh = _sc_mesh()
    NW = NC * NS
    assert total % NW == 0
    b_per_w = total // NW
    C = 256
    assert b_per_w % C == 0 and C % 16 == 0
    nchunk = b_per_w // C

    @functools.partial(
        pl.kernel,
        mesh=mesh,
        out_type=jax.ShapeDtypeStruct((total, D), jnp.float32),
        scratch_types=[
            pltpu.VMEM((b_per_w,), jnp.int32),
            pltpu.VMEM((b_per_w,), jnp.int32),
            pltpu.VMEM((b_per_w,), jnp.int32),
            pltpu.VMEM((2, C, 2 * D), jnp.float32),
            pltpu.VMEM((C, D), jnp.float32),
            pltpu.SemaphoreType.DMA,
            pltpu.SemaphoreType.DMA,
        ],
    )
    def gather_kernel(packed_hbm, idx_hbm, out_hbm,
                      idx_v, pr_v, off_v, pairs_v, sel_v, g0, g1):
        wid = lax.axis_index("s") * NC + lax.axis_index("c")
        base = wid * b_per_w
        pltpu.sync_copy(idx_hbm.at[pl.ds(base, b_per_w)], idx_v)

        def prep(i, _):
            v = idx_v[pl.ds(i * 16, 16)]
            pr_v[pl.ds(i * 16, 16)] = v >> 1
            off_v[pl.ds(i * 16, 16)] = (v & 1) * D
            return ()
        lax.fori_loop(0, b_per_w // 16, prep, ())

        gsems = (g0, g1)

        def start(c, b):
            pltpu.async_copy(
                packed_hbm.at[pr_v.at[pl.ds(c * C, C)]],
                pairs_v.at[b], gsems[b])

        def finish(c, b):
            pltpu.make_async_copy(
                packed_hbm.at[pl.ds(0, C), :], pairs_v.at[b], gsems[b]).wait()

            def select(g, _):
                offs = off_v[pl.ds(c * C + g * 16, 16)]
                for j in range(16):
                    o = offs[j]
                    for q in range(D // 16):
                        sel_v[g * 16 + j, pl.ds(q * 16, 16)] = (
                            pairs_v[b, g * 16 + j, pl.ds(o + q * 16, 16)])
                return ()
            lax.fori_loop(0, C // 16, select, ())
            pltpu.sync_copy(sel_v, out_hbm.at[pl.ds(base + c * C, C)])

        start(0, 0)

        def pair_body(c2, _):
            for b in range(2):
                c = c2 * 2 + b

                @pl.when(c + 1 < nchunk)
                def _():
                    start(c + 1, (b + 1) % 2)

                finish(c, b)
            return ()

        lax.fori_loop(0, nchunk // 2, pair_body, ())
        if nchunk % 2:
            finish(nchunk - 1, (nchunk - 1) % 2)

    return gather_kernel


def kernel(item_ids, item_codes):
    B, H = item_ids.shape
    N, D = item_codes.shape
    total = B * H
    flat_ids = item_ids.reshape(total).astype(jnp.int32)
    packed = _make_repack(N, D)(item_codes)
    out = _make_pair_gather(total, N, D)(packed, flat_ids)
    return out.reshape(B, H, D)


# speedup vs baseline: 1.3979x; 1.3103x over previous
import functools

import jax
import jax.numpy as jnp
from jax import lax
from jax.experimental import pallas as pl
from jax.experimental.pallas import tpu as pltpu
from jax.experimental.pallas import tpu_sc as plsc


def _sc_mesh():
    info = plsc.get_sparse_core_info()
    NC, NS = info.num_cores, info.num_subcores
    mesh = plsc.VectorSubcoreMesh(core_axis_name="c", subcore_axis_name="s")
    return NC, NS, mesh


def _make_repack(N: int, D: int):
    NC, NS, mesh = _sc_mesh()
    NW = NC * NS
    K = 400
    assert N % K == 0 and K % 16 == 0
    nchunks = N // K
    main = (nchunks // NW) & ~1
    extra = nchunks - main * NW

    @functools.partial(
        pl.kernel,
        mesh=mesh,
        out_type=jax.ShapeDtypeStruct((N // 2, 2 * D), jnp.float32),
        scratch_types=[
            pltpu.VMEM((2, K, D), jnp.float32),
            pltpu.VMEM((K // 2, 2 * D), jnp.float32),
            pltpu.SemaphoreType.DMA,
            pltpu.SemaphoreType.DMA,
        ],
    )
    def repack_kernel(table_hbm, out_hbm, buf, pbuf, s0, s1):
        wid = lax.axis_index("s") * NC + lax.axis_index("c")
        sems = (s0, s1)

        def start(cid, b):
            pltpu.async_copy(
                table_hbm.at[pl.ds(cid * K, K), :], buf.at[b], sems[b])

        def finish(cid, b):
            pltpu.make_async_copy(
                table_hbm.at[pl.ds(0, K), :], buf.at[b], sems[b]).wait()

            def pack(j, _):
                for p in range(2 * D // 16):
                    pbuf[j, pl.ds(p * 16, 16)] = (
                        buf[b, 2 * j + p * 16 // D, pl.ds(p * 16 % D, 16)])
                return ()
            lax.fori_loop(0, K // 2, pack, ())
            pltpu.sync_copy(
                pbuf, out_hbm.at[pl.ds(cid * (K // 2), K // 2), :])

        start(wid, 0)

        def pair_body(i2, _):
            for b in range(2):
                i = i2 * 2 + b

                @pl.when(i + 1 < main)
                def _():
                    start(wid + (i + 1) * NW, (b + 1) % 2)

                finish(wid + i * NW, b)
            return ()

        lax.fori_loop(0, main // 2, pair_body, ())

        @pl.when(wid < extra)
        def _():
            start(main * NW + wid, 0)
            finish(main * NW + wid, 0)

    return repack_kernel


def _make_pair_gather(total: int, N: int, D: int):
    NC, NS, mesh = _sc_mesh()
    NW = NC * NS
    assert total % NW == 0
    b_per_w = total // NW
    C = 256
    assert b_per_w % C == 0 and C % 16 == 0
    nchunk = b_per_w // C

    @functools.partial(
        pl.kernel,
        mesh=mesh,
        out_type=jax.ShapeDtypeStruct((total, D), jnp.float32),
        scratch_types=[
            pltpu.VMEM((b_per_w,), jnp.int32),
            pltpu.VMEM((b_per_w,), jnp.int32),
            pltpu.VMEM((b_per_w,), jnp.int32),
            pltpu.VMEM((2, C, 2 * D), jnp.float32),
            pltpu.VMEM((C, D), jnp.float32),
            pltpu.SemaphoreType.DMA,
            pltpu.SemaphoreType.DMA,
        ],
    )
    def gather_kernel(packed_hbm, idx_hbm, out_hbm,
                      idx_v, pr_v, off_v, pairs_v, sel_v, g0, g1):
        wid = lax.axis_index("s") * NC + lax.axis_index("c")
        base = wid * b_per_w
        pltpu.sync_copy(idx_hbm.at[pl.ds(base, b_per_w)], idx_v)

        def prep(i, _):
            v = idx_v[pl.ds(i * 16, 16)]
            pr_v[pl.ds(i * 16, 16)] = v >> 1
            off_v[pl.ds(i * 16, 16)] = (v & 1) * D
            return ()
        lax.fori_loop(0, b_per_w // 16, prep, ())

        gsems = (g0, g1)

        def start(c, b):
            pltpu.async_copy(
                packed_hbm.at[pr_v.at[pl.ds(c * C, C)]],
                pairs_v.at[b], gsems[b])

        def finish(c, b):
            pltpu.make_async_copy(
                packed_hbm.at[pl.ds(0, C), :], pairs_v.at[b], gsems[b]).wait()

            def select(g, _):
                offs = off_v[pl.ds(c * C + g * 16, 16)]
                for j in range(16):
                    o = offs[j]
                    for q in range(D // 16):
                        sel_v[g * 16 + j, pl.ds(q * 16, 16)] = (
                            pairs_v[b, g * 16 + j, pl.ds(o + q * 16, 16)])
                return ()
            lax.fori_loop(0, C // 16, select, ())
            pltpu.sync_copy(sel_v, out_hbm.at[pl.ds(base + c * C, C)])

        start(0, 0)

        def pair_body(c2, _):
            for b in range(2):
                c = c2 * 2 + b

                @pl.when(c + 1 < nchunk)
                def _():
                    start(c + 1, (b + 1) % 2)

                finish(c, b)
            return ()

        lax.fori_loop(0, nchunk // 2, pair_body, ())
        if nchunk % 2:
            finish(nchunk - 1, (nchunk - 1) % 2)

    return gather_kernel


def kernel(item_ids, item_codes):
    B, H = item_ids.shape
    N, D = item_codes.shape
    total = B * H
    flat_ids = item_ids.reshape(total).astype(jnp.int32)
    packed = item_codes.reshape(N // 2, 2 * D)
    out = _make_pair_gather(total, N, D)(packed, flat_ids)
    return out.reshape(B, H, D)


# speedup vs baseline: 1.4756x; 1.0556x over previous
import functools

import jax
import jax.numpy as jnp
from jax import lax
from jax.experimental import pallas as pl
from jax.experimental.pallas import tpu as pltpu
from jax.experimental.pallas import tpu_sc as plsc


def _make_pair_gather(total: int, N: int, D: int):
    info = plsc.get_sparse_core_info()
    NC, NS = info.num_cores, info.num_subcores
    mesh = plsc.VectorSubcoreMesh(core_axis_name="c", subcore_axis_name="s")
    NW = NC * NS
    assert total % NW == 0
    b_per_w = total // NW
    C = 256
    assert b_per_w % C == 0 and C % 16 == 0
    nchunk = b_per_w // C

    @functools.partial(
        pl.kernel,
        mesh=mesh,
        out_type=jax.ShapeDtypeStruct((total // 2, 2 * D), jnp.float32),
        scratch_types=[
            pltpu.VMEM((b_per_w,), jnp.int32),
            pltpu.VMEM((b_per_w,), jnp.int32),
            pltpu.VMEM((2, C, 2 * D), jnp.float32),
            pltpu.VMEM((2, C // 2, 2 * D), jnp.float32),
            pltpu.SemaphoreType.DMA,
            pltpu.SemaphoreType.DMA,
            pltpu.SemaphoreType.DMA,
            pltpu.SemaphoreType.DMA,
        ],
    )
    def gather_kernel(packed_hbm, idx_hbm, out_hbm,
                      pr_v, off_v, pairs_v, sel_v, g0, g1, w0, w1):
        wid = lax.axis_index("s") * NC + lax.axis_index("c")
        base = wid * b_per_w
        base2 = wid * (b_per_w // 2)
        pltpu.sync_copy(idx_hbm.at[pl.ds(base, b_per_w)], pr_v)

        def prep(i, _):
            v = pr_v[pl.ds(i * 16, 16)]
            pr_v[pl.ds(i * 16, 16)] = v >> 1
            off_v[pl.ds(i * 16, 16)] = (v & 1) * D
            return ()
        lax.fori_loop(0, b_per_w // 16, prep, ())

        gsems = (g0, g1)
        wsems = (w0, w1)

        def start(c, b):
            pltpu.async_copy(
                packed_hbm.at[pr_v.at[pl.ds(c * C, C)]],
                pairs_v.at[b], gsems[b])

        def drain_write(b):
            pltpu.make_async_copy(
                sel_v.at[b], out_hbm.at[pl.ds(0, C // 2)], wsems[b]).wait()

        def finish(c, b, first):
            pltpu.make_async_copy(
                packed_hbm.at[pl.ds(0, C), :], pairs_v.at[b], gsems[b]).wait()
            if not first:
                drain_write(b)

            def select(g, _):
                offs = off_v[pl.ds(c * C + g * 16, 16)]
                for j in range(16):
                    o = offs[j]
                    for q in range(D // 16):
                        sel_v[b, g * 8 + j // 2, pl.ds((j % 2) * D + q * 16, 16)] = (
                            pairs_v[b, g * 16 + j, pl.ds(o + q * 16, 16)])
                return ()
            lax.fori_loop(0, C // 16, select, ())
            pltpu.async_copy(
                sel_v.at[b],
                out_hbm.at[pl.ds(base2 + c * (C // 2), C // 2)], wsems[b])

        start(0, 0)
        start(1, 1)
        finish(0, 0, True)
        start(2, 0)
        finish(1, 1, True)

        def pair_body(c2, _):
            for b in range(2):
                c = c2 * 2 + b

                @pl.when(c + 1 < nchunk)
                def _():
                    start(c + 1, (b + 1) % 2)

                finish(c, b, False)
            return ()

        lax.fori_loop(1, nchunk // 2, pair_body, ())
        if nchunk % 2:
            finish(nchunk - 1, (nchunk - 1) % 2, False)
        drain_write(nchunk % 2)
        drain_write((nchunk + 1) % 2)

    return gather_kernel


def kernel(item_ids, item_codes):
    B, H = item_ids.shape
    N, D = item_codes.shape
    total = B * H
    flat_ids = item_ids.reshape(total).astype(jnp.int32)
    packed = item_codes.reshape(N // 2, 2 * D)
    out = _make_pair_gather(total, N, D)(packed, flat_ids)
    return out.reshape(B, H, D)


# speedup vs baseline: 1.5972x; 1.0824x over previous
import functools

import jax
import jax.numpy as jnp
from jax import lax
from jax.experimental import pallas as pl
from jax.experimental.pallas import tpu as pltpu
from jax.experimental.pallas import tpu_sc as plsc


def _make_gather(total: int, D: int):
    info = plsc.get_sparse_core_info()
    NS = info.num_subcores
    NW = NS
    assert total % NW == 0
    b_per_w = total // NW
    C = 800
    assert b_per_w % C == 0
    nchunk = b_per_w // C

    mesh = plsc.VectorSubcoreMesh(
        core_axis_name="c", subcore_axis_name="s", num_cores=1)

    @functools.partial(
        pl.kernel,
        mesh=mesh,
        out_type=jax.ShapeDtypeStruct((total, D), jnp.float32),
        scratch_types=[
            pltpu.VMEM((b_per_w,), jnp.int32),
            pltpu.VMEM((2, C, D), jnp.float32),
            pltpu.SemaphoreType.DMA,
            pltpu.SemaphoreType.DMA,
        ],
        compiler_params=pltpu.CompilerParams(use_tc_tiling_on_sc=False),
    )
    def gather_kernel(table_hbm, idx_hbm, out_hbm, idx_v, rows_v, g0, g1):
        wid = lax.axis_index("s")
        base = wid * b_per_w
        pltpu.sync_copy(idx_hbm.at[pl.ds(base, b_per_w)], idx_v)
        gsems = (g0, g1)

        def start(c, b):
            pltpu.async_copy(
                table_hbm.at[idx_v.at[pl.ds(c * C, C)]],
                rows_v.at[b], gsems[b])

        def finish(c, b):
            pltpu.make_async_copy(
                table_hbm.at[pl.ds(0, C), :], rows_v.at[b], gsems[b]).wait()
            pltpu.sync_copy(rows_v.at[b], out_hbm.at[pl.ds(base + c * C, C)])

        start(0, 0)

        def pair_body(c2, _):
            for b in range(2):
                c = c2 * 2 + b

                @pl.when(c + 1 < nchunk)
                def _():
                    start(c + 1, (b + 1) % 2)

                finish(c, b)
            return ()

        lax.fori_loop(0, nchunk // 2, pair_body, ())
        if nchunk % 2:
            finish(nchunk - 1, (nchunk - 1) % 2)

    return gather_kernel


def kernel(item_ids, item_codes):
    B, H = item_ids.shape
    N, D = item_codes.shape
    total = B * H
    flat_ids = item_ids.reshape(total).astype(jnp.int32)
    out = _make_gather(total, D)(item_codes, flat_ids)
    return out.reshape(B, H, D)


# speedup vs baseline: 1.6200x; 1.0143x over previous
import functools

import jax
import jax.numpy as jnp
from jax import lax
from jax.experimental import pallas as pl
from jax.experimental.pallas import tpu as pltpu
from jax.experimental.pallas import tpu_sc as plsc


def _make_gather(total: int, D: int):
    info = plsc.get_sparse_core_info()
    NC, NS = info.num_cores, info.num_subcores
    NW = NC * NS
    assert total % NW == 0
    b_per_w = total // NW
    C = 800
    assert b_per_w % C == 0
    nchunk = b_per_w // C

    mesh = plsc.VectorSubcoreMesh(core_axis_name="c", subcore_axis_name="s")

    @functools.partial(
        pl.kernel,
        mesh=mesh,
        out_type=jax.ShapeDtypeStruct((total, D), jnp.float32),
        scratch_types=[
            pltpu.VMEM((b_per_w,), jnp.int32),
            pltpu.VMEM((2, C, D), jnp.float32),
            pltpu.SemaphoreType.DMA,
            pltpu.SemaphoreType.DMA,
        ],
        compiler_params=pltpu.CompilerParams(
            use_tc_tiling_on_sc=False, skip_device_barrier=True),
    )
    def gather_kernel(table_hbm, idx_hbm, out_hbm, idx_v, rows_v, g0, g1):
        wid = lax.axis_index("s") * NC + lax.axis_index("c")
        base = wid * b_per_w
        pltpu.sync_copy(idx_hbm.at[pl.ds(base, b_per_w)], idx_v)
        gsems = (g0, g1)

        def start(c, b):
            pltpu.async_copy(
                table_hbm.at[idx_v.at[pl.ds(c * C, C)]],
                rows_v.at[b], gsems[b])

        def finish(c, b):
            pltpu.make_async_copy(
                table_hbm.at[pl.ds(0, C), :], rows_v.at[b], gsems[b]).wait()
            pltpu.sync_copy(rows_v.at[b], out_hbm.at[pl.ds(base + c * C, C)])

        start(0, 0)

        def pair_body(c2, _):
            for b in range(2):
                c = c2 * 2 + b

                @pl.when(c + 1 < nchunk)
                def _():
                    start(c + 1, (b + 1) % 2)

                finish(c, b)
            return ()

        lax.fori_loop(0, nchunk // 2, pair_body, ())
        if nchunk % 2:
            finish(nchunk - 1, (nchunk - 1) % 2)

    return gather_kernel


def kernel(item_ids, item_codes):
    B, H = item_ids.shape
    N, D = item_codes.shape
    total = B * H
    flat_ids = item_ids.reshape(total).astype(jnp.int32)
    out = _make_gather(total, D)(item_codes, flat_ids)
    return out.reshape(B, H, D)


# speedup vs baseline: 1.6202x; 1.0002x over previous
import functools

import jax
import jax.numpy as jnp
from jax import lax
from jax.experimental import pallas as pl
from jax.experimental.pallas import tpu as pltpu
from jax.experimental.pallas import tpu_sc as plsc


def _make_gather(total: int, D: int):
    info = plsc.get_sparse_core_info()
    NC, NS = info.num_cores, info.num_subcores
    NW = NC * NS
    assert total % NW == 0
    b_per_w = total // NW
    C = 800
    assert b_per_w % C == 0
    nchunk = b_per_w // C

    mesh = plsc.VectorSubcoreMesh(core_axis_name="c", subcore_axis_name="s")

    @functools.partial(
        pl.kernel,
        mesh=mesh,
        out_type=jax.ShapeDtypeStruct((total, D), jnp.float32),
        scratch_types=[
            pltpu.VMEM((b_per_w,), jnp.int32),
            pltpu.VMEM((2, C, D), jnp.float32),
            pltpu.SemaphoreType.DMA,
            pltpu.SemaphoreType.DMA,
        ],
        compiler_params=pltpu.CompilerParams(use_tc_tiling_on_sc=False),
    )
    def gather_kernel(table_hbm, idx_hbm, out_hbm, idx_v, rows_v, g0, g1):
        wid = lax.axis_index("s") * NC + lax.axis_index("c")
        base = wid * b_per_w
        pltpu.sync_copy(idx_hbm.at[pl.ds(base, b_per_w)], idx_v)
        gsems = (g0, g1)

        def start(c, b):
            pltpu.async_copy(
                table_hbm.at[idx_v.at[pl.ds(c * C, C)]],
                rows_v.at[b], gsems[b])

        def finish(c, b):
            pltpu.make_async_copy(
                table_hbm.at[pl.ds(0, C), :], rows_v.at[b], gsems[b]).wait()
            pltpu.sync_copy(rows_v.at[b], out_hbm.at[pl.ds(base + c * C, C)])

        start(0, 0)

        def pair_body(c2, _):
            for b in range(2):
                c = c2 * 2 + b

                @pl.when(c + 1 < nchunk)
                def _():
                    start(c + 1, (b + 1) % 2)

                finish(c, b)
            return ()

        lax.fori_loop(0, nchunk // 2, pair_body, ())
        if nchunk % 2:
            finish(nchunk - 1, (nchunk - 1) % 2)

    return gather_kernel


def kernel(item_ids, item_codes):
    B, H = item_ids.shape
    N, D = item_codes.shape
    total = B * H
    flat_ids = item_ids.reshape(total).astype(jnp.int32)
    out = _make_gather(total, D)(item_codes, flat_ids)
    return out.reshape(B, H, D)


# speedup vs baseline: 1.6218x; 1.0010x over previous
import functools

import jax
import jax.numpy as jnp
from jax import lax
from jax.experimental import pallas as pl
from jax.experimental.pallas import tpu as pltpu
from jax.experimental.pallas import tpu_sc as plsc


def _make_gather(total: int, D: int):
    info = plsc.get_sparse_core_info()
    NC, NS = info.num_cores, info.num_subcores
    NW = NC * NS
    assert total % NW == 0
    b_per_w = total // NW
    C = 800
    assert b_per_w % C == 0
    nchunk = b_per_w // C

    mesh = plsc.VectorSubcoreMesh(core_axis_name="c", subcore_axis_name="s")

    @functools.partial(
        pl.kernel,
        mesh=mesh,
        out_type=jax.ShapeDtypeStruct((total, D), jnp.float32),
        scratch_types=[
            pltpu.VMEM((b_per_w,), jnp.int32),
            pltpu.VMEM((2, C, D), jnp.float32),
            pltpu.SemaphoreType.DMA,
            pltpu.SemaphoreType.DMA,
            pltpu.SemaphoreType.DMA,
            pltpu.SemaphoreType.DMA,
        ],
        compiler_params=pltpu.CompilerParams(use_tc_tiling_on_sc=False),
    )
    def gather_kernel(table_hbm, idx_hbm, out_hbm, idx_v, rows_v,
                      g0, g1, w0, w1):
        wid = lax.axis_index("s") * NC + lax.axis_index("c")
        base = wid * b_per_w
        pltpu.sync_copy(idx_hbm.at[pl.ds(base, b_per_w)], idx_v)
        gsems = (g0, g1)
        wsems = (w0, w1)

        def start(c, b):
            pltpu.async_copy(
                table_hbm.at[idx_v.at[pl.ds(c * C, C)]],
                rows_v.at[b], gsems[b])

        def drain_write(b):
            pltpu.make_async_copy(
                rows_v.at[b], out_hbm.at[pl.ds(0, C)], wsems[b]).wait()

        start(0, 0)
        for c in range(nchunk):
            b = c % 2
            nb = (c + 1) % 2
            if c + 1 < nchunk:
                if c >= 1:
                    drain_write(nb)
                start(c + 1, nb)
            pltpu.make_async_copy(
                table_hbm.at[pl.ds(0, C), :], rows_v.at[b], gsems[b]).wait()
            pltpu.async_copy(
                rows_v.at[b], out_hbm.at[pl.ds(base + c * C, C)], wsems[b])
        drain_write(0)
        drain_write(1)

    return gather_kernel


def kernel(item_ids, item_codes):
    B, H = item_ids.shape
    N, D = item_codes.shape
    total = B * H
    flat_ids = item_ids.reshape(total).astype(jnp.int32)
    out = _make_gather(total, D)(item_codes, flat_ids)
    return out.reshape(B, H, D)
